# baseline (speedup 1.0000x reference)
import functools

import jax
import jax.numpy as jnp
from jax import lax
from jax.experimental import pallas as pl
from jax.experimental.pallas import tpu as pltpu
from jax.experimental.pallas import tpu_sc as plsc

B = 16384
F = 26
NW = 32
ROWS_PER_W = B // NW
SLAB = ROWS_PER_W * F
CHUNK = 128
NCHUNK = SLAB // CHUNK
NBLK = 4
VOCAB = 1000000
VPAD = 1000448
SEG_EVEN = 62496
SEG_ODD = 62504
BOUNCE = 12800


def _seg(k):
    base = (k // 2) * (SEG_EVEN + SEG_ODD) + (k % 2) * SEG_EVEN
    return base, (SEG_EVEN if k % 2 == 0 else SEG_ODD)


def _sc_body(w_hbm, idx_hbm, vals_hbm, b_hbm, out_hbm,
             spw, idx_v, vals_v, rows_v, out_v, b_v,
             s0, s1, s2, s3, vsem):
    cid = lax.axis_index("c")
    sid = lax.axis_index("s")
    wid = sid * 2 + cid
    sems = (s0, s1, s2, s3)

    icopy = pltpu.make_async_copy(idx_hbm.at[wid], idx_v, vsem)
    icopy.start()
    pltpu.sync_copy(b_hbm, b_v)
    icopy.wait()

    def fire(c, src):
        def body(f, carry):
            rr = f * NBLK + c
            pltpu.make_async_copy(
                src.at[idx_v.at[pl.ds(rr * CHUNK, CHUNK)]],
                rows_v.at[pl.ds(rr * CHUNK, CHUNK)], sems[c]).start()
            return carry
        lax.fori_loop(0, F, body, None, unroll=False)

    def drain(c):
        def body(f, carry):
            rr = f * NBLK + c
            pltpu.make_async_copy(
                spw.at[idx_v.at[pl.ds(rr * CHUNK, CHUNK)]],
                rows_v.at[pl.ds(rr * CHUNK, CHUNK)], sems[c]).wait()
            return carry
        lax.fori_loop(0, F, body, None, unroll=False)

    def compute(c):
        def body(d, carry):
            col = d * 16
            acc = b_v[...]
            for f in range(F):
                p = (f * NBLK + c) * CHUNK + col
                acc = acc + rows_v[pl.ds(p, 16)] * vals_v[pl.ds(p, 16)]
            out_v[pl.ds(c * CHUNK + col, 16)] = acc
            return carry
        lax.fori_loop(0, CHUNK // 16, body, None, unroll=False)

    fire(0, w_hbm)
    fire(1, w_hbm)

    for k in range(16):
        @pl.when(sid == k)
        def _(k=k):
            base, seglen = _seg(k)
            off = 0
            while off < seglen:
                n = min(BOUNCE, seglen - off)
                pltpu.sync_copy(w_hbm.at[pl.ds(base + off, n)],
                                vals_v.at[pl.ds(0, n)])
                pltpu.sync_copy(vals_v.at[pl.ds(0, n)],
                                spw.at[pl.ds(base + off, n)])
                off += n

    vcopy = pltpu.make_async_copy(vals_hbm.at[wid], vals_v, vsem)
    vcopy.start()
    plsc.subcore_barrier()

    fire(2, spw)
    drain(0)
    vcopy.wait()
    compute(0)
    fire(3, spw)
    drain(1)
    compute(1)
    drain(2)
    compute(2)
    drain(3)
    compute(3)

    pltpu.sync_copy(out_v, out_hbm.at[pl.ds(wid * ROWS_PER_W, ROWS_PER_W)])


@functools.partial(
    pl.kernel,
    out_type=jax.ShapeDtypeStruct((B,), jnp.float32),
    mesh=plsc.VectorSubcoreMesh(core_axis_name="c", subcore_axis_name="s"),
    scratch_types=[
        pltpu.VMEM_SHARED((VOCAB,), jnp.float32),
        pltpu.VMEM((SLAB,), jnp.int32),
        pltpu.VMEM((SLAB,), jnp.float32),
        pltpu.VMEM((SLAB,), jnp.float32),
        pltpu.VMEM((ROWS_PER_W,), jnp.float32),
        pltpu.VMEM((16,), jnp.float32),
        pltpu.SemaphoreType.DMA,
        pltpu.SemaphoreType.DMA,
        pltpu.SemaphoreType.DMA,
        pltpu.SemaphoreType.DMA,
        pltpu.SemaphoreType.DMA,
    ],
)
def _lr_sc_kernel(w_hbm, idx_hbm, vals_hbm, b_hbm, out_hbm,
                  spw, idx_v, vals_v, rows_v, out_v, b_v,
                  s0, s1, s2, s3, vsem):
    _sc_body(w_hbm, idx_hbm, vals_hbm, b_hbm, out_hbm,
             spw, idx_v, vals_v, rows_v, out_v, b_v,
             s0, s1, s2, s3, vsem)


def kernel(indices, values, w, b):
    idx_r = (indices.astype(jnp.int32)
             .reshape(NW, ROWS_PER_W, F)
             .transpose(0, 2, 1)
             .reshape(NW, SLAB))
    vals_r = (values
              .reshape(NW, ROWS_PER_W, F)
              .transpose(0, 2, 1)
              .reshape(NW, SLAB))
    w_flat = jax.lax.dynamic_update_slice(
        jnp.zeros((VPAD, 1), jnp.float32), w, (0, 0)).reshape(-1)
    b16 = jnp.broadcast_to(b.astype(jnp.float32), (16,))
    out = _lr_sc_kernel(w_flat, idx_r, vals_r, b16)
    return out.reshape(B, 1)


# speedup vs baseline: 1.0075x; 1.0075x over previous
import functools

import jax
import jax.numpy as jnp
from jax import lax
from jax.experimental import pallas as pl
from jax.experimental.pallas import tpu as pltpu
from jax.experimental.pallas import tpu_sc as plsc

B = 16384
F = 26
NW = 32
ROWS_PER_W = B // NW
SLAB = ROWS_PER_W * F
VOCAB = 1000000
VPAD = 1000448
SEG_EVEN = 62496
SEG_ODD = 62504
BOUNCE = 12800


def _seg(k):
    base = (k // 2) * (SEG_EVEN + SEG_ODD) + (k % 2) * SEG_EVEN
    return base, (SEG_EVEN if k % 2 == 0 else SEG_ODD)


def _sc_body(w_hbm, idx_hbm, vals_hbm, b_hbm, out_hbm,
             spw, idx_v, vals_v, rows_v, out_v, b_v, sem, vsem):
    cid = lax.axis_index("c")
    sid = lax.axis_index("s")
    wid = sid * 2 + cid

    icopy = pltpu.make_async_copy(idx_hbm.at[wid], idx_v, vsem)
    icopy.start()
    vcopy = pltpu.make_async_copy(vals_hbm.at[wid], vals_v, vsem)
    vcopy.start()
    pltpu.sync_copy(b_hbm, b_v)

    for k in range(16):
        @pl.when(sid == k)
        def _(k=k):
            base, seglen = _seg(k)
            off = 0
            while off < seglen:
                n = min(BOUNCE, seglen - off)
                pltpu.sync_copy(w_hbm.at[pl.ds(base + off, n)],
                                rows_v.at[pl.ds(0, n)])
                pltpu.sync_copy(rows_v.at[pl.ds(0, n)],
                                spw.at[pl.ds(base + off, n)])
                off += n

    icopy.wait()
    vcopy.wait()
    plsc.subcore_barrier()

    copies = [
        pltpu.make_async_copy(
            spw.at[idx_v.at[pl.ds(f * ROWS_PER_W, ROWS_PER_W)]],
            rows_v.at[pl.ds(f * ROWS_PER_W, ROWS_PER_W)], sem)
        for f in range(F)
    ]
    for cp in copies:
        cp.start()
    for cp in copies:
        cp.wait()

    for g in range(ROWS_PER_W // 16):
        col = g * 16
        acc = b_v[...]
        for f in range(F):
            p = f * ROWS_PER_W + col
            acc = acc + rows_v[pl.ds(p, 16)] * vals_v[pl.ds(p, 16)]
        out_v[pl.ds(col, 16)] = acc

    pltpu.sync_copy(out_v, out_hbm.at[pl.ds(wid * ROWS_PER_W, ROWS_PER_W)])


@functools.partial(
    pl.kernel,
    out_type=jax.ShapeDtypeStruct((B,), jnp.float32),
    mesh=plsc.VectorSubcoreMesh(core_axis_name="c", subcore_axis_name="s"),
    scratch_types=[
        pltpu.VMEM_SHARED((VOCAB,), jnp.float32),
        pltpu.VMEM((SLAB,), jnp.int32),
        pltpu.VMEM((SLAB,), jnp.float32),
        pltpu.VMEM((SLAB,), jnp.float32),
        pltpu.VMEM((ROWS_PER_W,), jnp.float32),
        pltpu.VMEM((16,), jnp.float32),
        pltpu.SemaphoreType.DMA,
        pltpu.SemaphoreType.DMA,
    ],
)
def _lr_sc_kernel(w_hbm, idx_hbm, vals_hbm, b_hbm, out_hbm,
                  spw, idx_v, vals_v, rows_v, out_v, b_v, sem, vsem):
    _sc_body(w_hbm, idx_hbm, vals_hbm, b_hbm, out_hbm,
             spw, idx_v, vals_v, rows_v, out_v, b_v, sem, vsem)


def kernel(indices, values, w, b):
    idx_r = (indices.astype(jnp.int32)
             .reshape(NW, ROWS_PER_W, F)
             .transpose(0, 2, 1)
             .reshape(NW, SLAB))
    vals_r = (values
              .reshape(NW, ROWS_PER_W, F)
              .transpose(0, 2, 1)
              .reshape(NW, SLAB))
    w_flat = jax.lax.dynamic_update_slice(
        jnp.zeros((VPAD, 1), jnp.float32), w, (0, 0)).reshape(-1)
    b16 = jnp.broadcast_to(b.astype(jnp.float32), (16,))
    out = _lr_sc_kernel(w_flat, idx_r, vals_r, b16)
    return out.reshape(B, 1)


# speedup vs baseline: 1.1178x; 1.1094x over previous
import functools

import jax
import jax.numpy as jnp
from jax import lax
from jax.experimental import pallas as pl
from jax.experimental.pallas import tpu as pltpu
from jax.experimental.pallas import tpu_sc as plsc

B = 16384
F = 26
NW = 32
ROWS_PER_W = B // NW
SLAB = ROWS_PER_W * F
VOCAB = 1000000
VPAD = 1000448
SEG_EVEN = 62496
SEG_ODD = 62504
BOUNCE = 12800


def _seg(k):
    base = (k // 2) * (SEG_EVEN + SEG_ODD) + (k % 2) * SEG_EVEN
    return base, (SEG_EVEN if k % 2 == 0 else SEG_ODD)


def _sc_body(w_hbm, idx_hbm, vals_hbm, b_hbm, out_hbm,
             spw, idx_v, vals_v, rows_v, out_v, b_v, sem, vsem):
    cid = lax.axis_index("c")
    sid = lax.axis_index("s")
    wid = sid * 2 + cid
    base = wid * ROWS_PER_W

    slabs = []
    for f in range(F):
        slabs.append(pltpu.make_async_copy(
            idx_hbm.at[f, pl.ds(base, ROWS_PER_W)],
            idx_v.at[pl.ds(f * ROWS_PER_W, ROWS_PER_W)], vsem))
        slabs.append(pltpu.make_async_copy(
            vals_hbm.at[f, pl.ds(base, ROWS_PER_W)],
            vals_v.at[pl.ds(f * ROWS_PER_W, ROWS_PER_W)], vsem))
    for cp in slabs:
        cp.start()
    pltpu.sync_copy(b_hbm, b_v)

    for k in range(16):
        @pl.when(sid == k)
        def _(k=k):
            sbase, seglen = _seg(k)
            off = 0
            while off < seglen:
                n = min(BOUNCE, seglen - off)
                pltpu.sync_copy(w_hbm.at[pl.ds(sbase + off, n)],
                                rows_v.at[pl.ds(0, n)])
                pltpu.sync_copy(rows_v.at[pl.ds(0, n)],
                                spw.at[pl.ds(sbase + off, n)])
                off += n

    for cp in slabs:
        cp.wait()
    plsc.subcore_barrier()

    copies = [
        pltpu.make_async_copy(
            spw.at[idx_v.at[pl.ds(f * ROWS_PER_W, ROWS_PER_W)]],
            rows_v.at[pl.ds(f * ROWS_PER_W, ROWS_PER_W)], sem)
        for f in range(F)
    ]
    for cp in copies:
        cp.start()
    for cp in copies:
        cp.wait()

    for g in range(ROWS_PER_W // 16):
        col = g * 16
        acc = b_v[...]
        for f in range(F):
            p = f * ROWS_PER_W + col
            acc = acc + rows_v[pl.ds(p, 16)] * vals_v[pl.ds(p, 16)]
        out_v[pl.ds(col, 16)] = acc

    pltpu.sync_copy(out_v, out_hbm.at[pl.ds(base, ROWS_PER_W)])


@functools.partial(
    pl.kernel,
    out_type=jax.ShapeDtypeStruct((B,), jnp.float32),
    mesh=plsc.VectorSubcoreMesh(core_axis_name="c", subcore_axis_name="s"),
    scratch_types=[
        pltpu.VMEM_SHARED((VOCAB,), jnp.float32),
        pltpu.VMEM((SLAB,), jnp.int32),
        pltpu.VMEM((SLAB,), jnp.float32),
        pltpu.VMEM((SLAB,), jnp.float32),
        pltpu.VMEM((ROWS_PER_W,), jnp.float32),
        pltpu.VMEM((16,), jnp.float32),
        pltpu.SemaphoreType.DMA,
        pltpu.SemaphoreType.DMA,
    ],
)
def _lr_sc_kernel(w_hbm, idx_hbm, vals_hbm, b_hbm, out_hbm,
                  spw, idx_v, vals_v, rows_v, out_v, b_v, sem, vsem):
    _sc_body(w_hbm, idx_hbm, vals_hbm, b_hbm, out_hbm,
             spw, idx_v, vals_v, rows_v, out_v, b_v, sem, vsem)


def kernel(indices, values, w, b):
    idx_t = indices.astype(jnp.int32).T
    vals_t = values.T
    w_flat = jax.lax.dynamic_update_slice(
        jnp.zeros((VPAD, 1), jnp.float32), w, (0, 0)).reshape(-1)
    b16 = jnp.broadcast_to(b.astype(jnp.float32), (16,))
    out = _lr_sc_kernel(w_flat, idx_t, vals_t, b16)
    return out.reshape(B, 1)


# speedup vs baseline: 1.1736x; 1.0500x over previous
import functools

import jax
import jax.numpy as jnp
from jax import lax
from jax.experimental import pallas as pl
from jax.experimental.pallas import tpu as pltpu
from jax.experimental.pallas import tpu_sc as plsc

B = 16384
F = 26
NW = 32
ROWS_PER_W = B // NW
SLAB = ROWS_PER_W * F
VOCAB = 1000000
VPAD = 1000448
SEG_EVEN = 62496
SEG_ODD = 62504
BOUNCE = 6400


def _seg(k):
    base = (k // 2) * (SEG_EVEN + SEG_ODD) + (k % 2) * SEG_EVEN
    return base, (SEG_EVEN if k % 2 == 0 else SEG_ODD)


def _sc_body(w_hbm, idx_hbm, vals_hbm, b_hbm, out_hbm,
             spw, idx_v, vals_v, rows_v, out_v, b_v, sem, vsem, osem):
    cid = lax.axis_index("c")
    sid = lax.axis_index("s")
    wid = sid * 2 + cid
    base = wid * ROWS_PER_W

    slabs = []
    for f in range(F):
        slabs.append(pltpu.make_async_copy(
            idx_hbm.at[f, pl.ds(base, ROWS_PER_W)],
            idx_v.at[pl.ds(f * ROWS_PER_W, ROWS_PER_W)], vsem))
        slabs.append(pltpu.make_async_copy(
            vals_hbm.at[f, pl.ds(base, ROWS_PER_W)],
            vals_v.at[pl.ds(f * ROWS_PER_W, ROWS_PER_W)], vsem))
    for cp in slabs:
        cp.start()
    pltpu.sync_copy(b_hbm, b_v)

    bufs = (rows_v.at[pl.ds(0, BOUNCE)], rows_v.at[pl.ds(6656, BOUNCE)])
    for k in range(16):
        @pl.when(sid == k)
        def _(k=k):
            sbase, seglen = _seg(k)
            chunks = []
            off = 0
            while off < seglen:
                n = min(BOUNCE, seglen - off)
                chunks.append((off, n))
                off += n
            outs = [None, None]
            for j, (coff, n) in enumerate(chunks):
                buf = bufs[j % 2]
                if outs[j % 2] is not None:
                    outs[j % 2].wait()
                pltpu.sync_copy(w_hbm.at[pl.ds(sbase + coff, n)],
                                buf.at[pl.ds(0, n)])
                cp = pltpu.make_async_copy(
                    buf.at[pl.ds(0, n)],
                    spw.at[pl.ds(sbase + coff, n)], osem)
                cp.start()
                outs[j % 2] = cp
            for cp in outs:
                if cp is not None:
                    cp.wait()

    for cp in slabs:
        cp.wait()
    plsc.subcore_barrier()

    copies = [
        pltpu.make_async_copy(
            spw.at[idx_v.at[pl.ds(f * ROWS_PER_W, ROWS_PER_W)]],
            rows_v.at[pl.ds(f * ROWS_PER_W, ROWS_PER_W)], sem)
        for f in range(F)
    ]
    for cp in copies:
        cp.start()
    for cp in copies:
        cp.wait()

    def compute_group(g, carry):
        col = g * 16
        acc = b_v[...]
        for f in range(F):
            p = f * ROWS_PER_W + col
            acc = acc + rows_v[pl.ds(p, 16)] * vals_v[pl.ds(p, 16)]
        out_v[pl.ds(col, 16)] = acc
        return carry

    lax.fori_loop(0, ROWS_PER_W // 16, compute_group, None, unroll=False)

    pltpu.sync_copy(out_v, out_hbm.at[pl.ds(base, ROWS_PER_W)])


@functools.partial(
    pl.kernel,
    out_type=jax.ShapeDtypeStruct((B,), jnp.float32),
    mesh=plsc.VectorSubcoreMesh(core_axis_name="c", subcore_axis_name="s"),
    scratch_types=[
        pltpu.VMEM_SHARED((VOCAB,), jnp.float32),
        pltpu.VMEM((SLAB,), jnp.int32),
        pltpu.VMEM((SLAB,), jnp.float32),
        pltpu.VMEM((SLAB,), jnp.float32),
        pltpu.VMEM((ROWS_PER_W,), jnp.float32),
        pltpu.VMEM((16,), jnp.float32),
        pltpu.SemaphoreType.DMA,
        pltpu.SemaphoreType.DMA,
        pltpu.SemaphoreType.DMA,
    ],
)
def _lr_sc_kernel(w_hbm, idx_hbm, vals_hbm, b_hbm, out_hbm,
                  spw, idx_v, vals_v, rows_v, out_v, b_v, sem, vsem, osem):
    _sc_body(w_hbm, idx_hbm, vals_hbm, b_hbm, out_hbm,
             spw, idx_v, vals_v, rows_v, out_v, b_v, sem, vsem, osem)


def kernel(indices, values, w, b):
    idx_t = indices.astype(jnp.int32).T
    vals_t = values.T
    w_flat = jax.lax.dynamic_update_slice(
        jnp.zeros((VPAD, 1), jnp.float32), w, (0, 0)).reshape(-1)
    b16 = jnp.broadcast_to(b.astype(jnp.float32), (16,))
    out = _lr_sc_kernel(w_flat, idx_t, vals_t, b16)
    return out.reshape(B, 1)


# speedup vs baseline: 1.1844x; 1.0091x over previous
import functools

import jax
import jax.numpy as jnp
from jax import lax
from jax.experimental import pallas as pl
from jax.experimental.pallas import tpu as pltpu
from jax.experimental.pallas import tpu_sc as plsc

B = 16384
F = 26
NW = 32
ROWS_PER_W = B // NW
SLAB = ROWS_PER_W * F
VOCAB = 1000000
VPAD = 1000448
SEG_EVEN = 62496
SEG_ODD = 62504
BOUNCE = 6400


def _seg(k):
    base = (k // 2) * (SEG_EVEN + SEG_ODD) + (k % 2) * SEG_EVEN
    return base, (SEG_EVEN if k % 2 == 0 else SEG_ODD)


def _sc_body(w_hbm, idx_hbm, vals_hbm, b_hbm, out_hbm,
             spw, idx_v, vals_v, rows_v, out_v, b_v, sem, vsem, osem):
    cid = lax.axis_index("c")
    sid = lax.axis_index("s")
    wid = sid * 2 + cid
    base = wid * ROWS_PER_W

    def slab_start(f, carry):
        pltpu.make_async_copy(
            idx_hbm.at[f, pl.ds(base, ROWS_PER_W)],
            idx_v.at[pl.ds(f * ROWS_PER_W, ROWS_PER_W)], vsem).start()
        pltpu.make_async_copy(
            vals_hbm.at[f, pl.ds(base, ROWS_PER_W)],
            vals_v.at[pl.ds(f * ROWS_PER_W, ROWS_PER_W)], vsem).start()
        return carry

    lax.fori_loop(0, F, slab_start, None, unroll=False)
    pltpu.sync_copy(b_hbm, b_v)

    bufs = (rows_v.at[pl.ds(0, BOUNCE)], rows_v.at[pl.ds(6656, BOUNCE)])
    for k in range(16):
        @pl.when(sid == k)
        def _(k=k):
            sbase, seglen = _seg(k)
            chunks = []
            off = 0
            while off < seglen:
                n = min(BOUNCE, seglen - off)
                chunks.append((off, n))
                off += n
            outs = [None, None]
            for j, (coff, n) in enumerate(chunks):
                buf = bufs[j % 2]
                if outs[j % 2] is not None:
                    outs[j % 2].wait()
                pltpu.sync_copy(w_hbm.at[pl.ds(sbase + coff, n)],
                                buf.at[pl.ds(0, n)])
                cp = pltpu.make_async_copy(
                    buf.at[pl.ds(0, n)],
                    spw.at[pl.ds(sbase + coff, n)], osem)
                cp.start()
                outs[j % 2] = cp
            for cp in outs:
                if cp is not None:
                    cp.wait()

    def slab_wait(f, carry):
        pltpu.make_async_copy(
            idx_hbm.at[f, pl.ds(base, ROWS_PER_W)],
            idx_v.at[pl.ds(f * ROWS_PER_W, ROWS_PER_W)], vsem).wait()
        pltpu.make_async_copy(
            vals_hbm.at[f, pl.ds(base, ROWS_PER_W)],
            vals_v.at[pl.ds(f * ROWS_PER_W, ROWS_PER_W)], vsem).wait()
        return carry

    lax.fori_loop(0, F, slab_wait, None, unroll=False)
    plsc.subcore_barrier()

    def gather_start(f, carry):
        pltpu.make_async_copy(
            spw.at[idx_v.at[pl.ds(f * ROWS_PER_W, ROWS_PER_W)]],
            rows_v.at[pl.ds(f * ROWS_PER_W, ROWS_PER_W)], sem).start()
        return carry

    def gather_wait(f, carry):
        pltpu.make_async_copy(
            spw.at[idx_v.at[pl.ds(f * ROWS_PER_W, ROWS_PER_W)]],
            rows_v.at[pl.ds(f * ROWS_PER_W, ROWS_PER_W)], sem).wait()
        return carry

    lax.fori_loop(0, F, gather_start, None, unroll=False)
    lax.fori_loop(0, F, gather_wait, None, unroll=False)

    def compute_group(g, carry):
        col = g * 16
        acc = b_v[...]
        for f in range(F):
            p = f * ROWS_PER_W + col
            acc = acc + rows_v[pl.ds(p, 16)] * vals_v[pl.ds(p, 16)]
        out_v[pl.ds(col, 16)] = acc
        return carry

    lax.fori_loop(0, ROWS_PER_W // 16, compute_group, None, unroll=False)

    pltpu.sync_copy(out_v, out_hbm.at[pl.ds(base, ROWS_PER_W)])


@functools.partial(
    pl.kernel,
    out_type=jax.ShapeDtypeStruct((B,), jnp.float32),
    mesh=plsc.VectorSubcoreMesh(core_axis_name="c", subcore_axis_name="s"),
    scratch_types=[
        pltpu.VMEM_SHARED((VOCAB,), jnp.float32),
        pltpu.VMEM((SLAB,), jnp.int32),
        pltpu.VMEM((SLAB,), jnp.float32),
        pltpu.VMEM((SLAB,), jnp.float32),
        pltpu.VMEM((ROWS_PER_W,), jnp.float32),
        pltpu.VMEM((16,), jnp.float32),
        pltpu.SemaphoreType.DMA,
        pltpu.SemaphoreType.DMA,
        pltpu.SemaphoreType.DMA,
    ],
)
def _lr_sc_kernel(w_hbm, idx_hbm, vals_hbm, b_hbm, out_hbm,
                  spw, idx_v, vals_v, rows_v, out_v, b_v, sem, vsem, osem):
    _sc_body(w_hbm, idx_hbm, vals_hbm, b_hbm, out_hbm,
             spw, idx_v, vals_v, rows_v, out_v, b_v, sem, vsem, osem)


def kernel(indices, values, w, b):
    idx_t = indices.astype(jnp.int32).T
    vals_t = values.T
    w_flat = jax.lax.dynamic_update_slice(
        jnp.zeros((VPAD, 1), jnp.float32), w, (0, 0)).reshape(-1)
    b16 = jnp.broadcast_to(b.astype(jnp.float32), (16,))
    out = _lr_sc_kernel(w_flat, idx_t, vals_t, b16)
    return out.reshape(B, 1)
